# baseline (speedup 1.0000x reference)
import jax
import jax.numpy as jnp
from jax.experimental import pallas as pl
from jax.experimental.pallas import tpu as pltpu

BLK = 256


def _cumsum_body(ns_ref, out_ref, carry_ref):
    i = pl.program_id(0)

    @pl.when(i == 0)
    def _():
        carry_ref[...] = jnp.zeros_like(carry_ref)

    blk = ns_ref[...]
    r = jax.lax.broadcasted_iota(jnp.int32, (BLK, BLK), 0)
    c = jax.lax.broadcasted_iota(jnp.int32, (BLK, BLK), 1)
    tri = (r >= c).astype(jnp.float32)
    cs = jnp.dot(tri, blk, preferred_element_type=jnp.float32) + carry_ref[...]
    out_ref[...] = cs
    carry_ref[...] = cs[BLK - 1:BLK, :]


def kernel(x, edge_index, edge_attr):
    u, v = edge_index[0], edge_index[1]
    e = edge_index.shape[1]
    n = x.shape[0]
    d = x.shape[1]
    nodes2 = jnp.concatenate([u, v])
    eids = jnp.concatenate([jnp.arange(e, dtype=jnp.int32)] * 2)
    order = jnp.lexsort((eids, nodes2))
    out_edge_attr = jnp.take(edge_attr, eids[order], axis=0)

    deg = jnp.zeros((n,), jnp.float32).at[nodes2].add(1.0)
    deg2 = jnp.maximum(deg * deg, 1.0)
    ns = deg2[:, None] * x

    n_pad = ((n + BLK - 1) // BLK) * BLK
    ns_p = jnp.pad(ns, ((0, n_pad - n), (0, 0)))
    cs = pl.pallas_call(
        _cumsum_body,
        grid=(n_pad // BLK,),
        in_specs=[pl.BlockSpec((BLK, d), lambda i: (i, 0))],
        out_specs=pl.BlockSpec((BLK, d), lambda i: (i, 0)),
        out_shape=jax.ShapeDtypeStruct((n_pad, d), jnp.float32),
        scratch_shapes=[pltpu.VMEM((1, d), jnp.float32)],
    )(ns_p)
    out_x = cs[:n] / deg2[:, None]
    return (out_x, out_edge_attr)


# speedup vs baseline: 2.7674x; 2.7674x over previous
import functools

import jax
import jax.numpy as jnp
from jax import lax
from jax.experimental import pallas as pl
from jax.experimental.pallas import tpu as pltpu
from jax.experimental.pallas import tpu_sc as plsc

N = 10000
E = 160000
S = 2 * E
D_EDGE = 16
NW = 32
PW = S // NW
NA, NB = 80, 128
CH = 2000
BLK = 256

_mesh = plsc.VectorSubcoreMesh(core_axis_name="c", subcore_axis_name="s")
_sc_params = pltpu.CompilerParams(needs_layout_passes=False)


def _wid():
    return lax.axis_index("s") * 2 + lax.axis_index("c")


@functools.partial(
    pl.kernel, mesh=_mesh,
    out_type=jax.ShapeDtypeStruct((NW, NA, NB), jnp.int32),
    compiler_params=_sc_params,
    scratch_types=[
        pltpu.VMEM((PW,), jnp.int32),
        pltpu.VMEM((NA, NB), jnp.int32),
    ],
)
def _k1_hist(nodes_hbm, hist_hbm, nd_v, h_v):
    w = _wid()
    pltpu.sync_copy(nodes_hbm.at[pl.ds(w * PW, PW)], nd_v)
    zeros = jnp.zeros((16,), jnp.int32)

    def zbody(m, _):
        h_v[m >> 3, pl.ds((m & 7) * 16, 16)] = zeros
        return 0

    lax.fori_loop(0, NA * 8, zbody, 0, unroll=4)
    ones = jnp.ones((16,), jnp.int32)

    def body(t, _):
        nd = nd_v[pl.ds(t * 16, 16)]
        plsc.addupdate_scatter(h_v.at[:, :], [nd >> 7, nd & 127], ones)
        return 0

    lax.fori_loop(0, PW // 16, body, 0, unroll=4)
    pltpu.sync_copy(h_v, hist_hbm.at[w])


def _k2a_body(h_ref, bases_ref, deg2_ref, acc, offs, deg2s):
    i = pl.program_id(0)

    @pl.when(i == 0)
    def _():
        acc[...] = jnp.zeros_like(acc)
        offs[...] = jnp.zeros_like(offs)
        deg2s[...] = jnp.zeros_like(deg2s)

    h = h_ref[0].astype(jnp.float32)

    @pl.when(i < NW)
    def _():
        bases_ref[0] = jnp.zeros((NA, NB), jnp.int32)
        acc[...] = acc[...] + h

        @pl.when(i == NW - 1)
        def _():
            a = acc[...]
            r1 = lax.broadcasted_iota(jnp.int32, (NB, NB), 0)
            c1 = lax.broadcasted_iota(jnp.int32, (NB, NB), 1)
            tri_u = (r1 <= c1).astype(jnp.float32)
            incl = jnp.dot(a, tri_u, preferred_element_type=jnp.float32, precision=lax.Precision.HIGHEST)
            rowsum = jnp.sum(a, axis=1, keepdims=True)
            r2 = lax.broadcasted_iota(jnp.int32, (NA, NA), 0)
            c2 = lax.broadcasted_iota(jnp.int32, (NA, NA), 1)
            tri_l = (r2 > c2).astype(jnp.float32)
            rowpre = jnp.dot(tri_l, rowsum, preferred_element_type=jnp.float32, precision=lax.Precision.HIGHEST)
            offs[...] = incl - a + rowpre
            deg2s[...] = jnp.maximum(a * a, 1.0)
            acc[...] = jnp.zeros_like(acc)

    @pl.when(i >= NW)
    def _():
        bases_ref[0] = (offs[...] + acc[...]).astype(jnp.int32)
        acc[...] = acc[...] + h

    deg2_ref[...] = deg2s[...]


def _k2a(hist):
    return pl.pallas_call(
        _k2a_body,
        grid=(2 * NW,),
        in_specs=[pl.BlockSpec((1, NA, NB), lambda i: (i % NW, 0, 0))],
        out_specs=[
            pl.BlockSpec((1, NA, NB), lambda i: (i % NW, 0, 0)),
            pl.BlockSpec((NA, NB), lambda i: (0, 0)),
        ],
        out_shape=[
            jax.ShapeDtypeStruct((NW, NA, NB), jnp.int32),
            jax.ShapeDtypeStruct((NA, NB), jnp.float32),
        ],
        scratch_shapes=[
            pltpu.VMEM((NA, NB), jnp.float32),
            pltpu.VMEM((NA, NB), jnp.float32),
            pltpu.VMEM((NA, NB), jnp.float32),
        ],
    )(hist)


def _k2c_body(ns_ref, out_ref, carry_ref):
    i = pl.program_id(0)

    @pl.when(i == 0)
    def _():
        carry_ref[...] = jnp.zeros_like(carry_ref)

    blk = ns_ref[...]
    r = lax.broadcasted_iota(jnp.int32, (BLK, BLK), 0)
    c = lax.broadcasted_iota(jnp.int32, (BLK, BLK), 1)
    tri = (r >= c).astype(jnp.float32)
    cs = jnp.dot(tri, blk, preferred_element_type=jnp.float32, precision=lax.Precision.HIGHEST) + carry_ref[...]
    out_ref[...] = cs
    carry_ref[...] = cs[BLK - 1:BLK, :]


def _k2c_cumsum(ns):
    n_pad, d = ns.shape
    return pl.pallas_call(
        _k2c_body,
        grid=(n_pad // BLK,),
        in_specs=[pl.BlockSpec((BLK, d), lambda i: (i, 0))],
        out_specs=pl.BlockSpec((BLK, d), lambda i: (i, 0)),
        out_shape=jax.ShapeDtypeStruct((n_pad, d), jnp.float32),
        scratch_shapes=[pltpu.VMEM((1, d), jnp.float32)],
    )(ns)


@functools.partial(
    pl.kernel, mesh=_mesh,
    out_type=jax.ShapeDtypeStruct((S,), jnp.int32),
    compiler_params=_sc_params,
    scratch_types=[
        pltpu.VMEM((PW,), jnp.int32),
        pltpu.VMEM((NA, NB), jnp.int32),
        pltpu.VMEM((PW,), jnp.int32),
        pltpu.VMEM((PW,), jnp.int32),
        pltpu.SemaphoreType.DMA,
    ],
)
def _k3_rank(nodes_hbm, bases_hbm, perm_hbm, nd_v, cnt_v, pos_v, eid_v, sem):
    w = _wid()
    pltpu.sync_copy(nodes_hbm.at[pl.ds(w * PW, PW)], nd_v)
    pltpu.sync_copy(bases_hbm.at[w], cnt_v)
    lanes = lax.iota(jnp.int32, 16)
    g0 = w * PW

    def body(t, _):
        sl = pl.ds(t * 16, 16)
        nd = nd_v[sl]
        hi = nd >> 7
        lo = nd & 127
        occ, last = plsc.scan_count(nd)
        base = plsc.load_gather(cnt_v.at[:, :], [hi, lo])
        pos_v[sl] = base + occ - 1
        plsc.store_scatter(cnt_v.at[:, :], [hi, lo], base + occ, mask=last)
        eid_v[sl] = (g0 + t * 16 + lanes) >> 1
        return 0

    lax.fori_loop(0, PW // 16, body, 0, unroll=2)
    pltpu.async_copy(eid_v, perm_hbm.at[pos_v], sem).wait()


@functools.partial(
    pl.kernel, mesh=_mesh,
    out_type=jax.ShapeDtypeStruct((S * D_EDGE,), jnp.float32),
    compiler_params=_sc_params,
    scratch_types=[
        pltpu.VMEM((PW,), jnp.int32),
        pltpu.VMEM((CH * D_EDGE,), jnp.int32),
        pltpu.VMEM((CH * D_EDGE,), jnp.float32),
        pltpu.SemaphoreType.DMA,
    ],
)
def _k4_gather(attr_hbm, perm_hbm, out_hbm, perm_v, idx_v, rows_v, sem):
    w = _wid()
    base = w * PW
    pltpu.sync_copy(perm_hbm.at[pl.ds(base, PW)], perm_v)
    lanes = lax.iota(jnp.int32, 16)

    def chunk_body(c, _):
        c0 = c * CH

        def row_body(j, _):
            p16 = perm_v[pl.ds(c0 + j * 16, 16)] * 16
            tgt = (lanes + j * 16) * 16
            for l in range(16):
                plsc.store_scatter(idx_v.at[:], [tgt + l], p16 + l)
            return 0

        lax.fori_loop(0, CH // 16, row_body, 0, unroll=2)
        pltpu.async_copy(attr_hbm.at[idx_v], rows_v, sem).wait()
        pltpu.sync_copy(rows_v, out_hbm.at[pl.ds((base + c0) * D_EDGE, CH * D_EDGE)])
        return 0

    lax.fori_loop(0, PW // CH, chunk_body, 0)


def kernel(x, edge_index, edge_attr):
    nodes_int = edge_index.astype(jnp.int32).T.reshape(-1)
    attr_flat = edge_attr.reshape(-1)

    hist = _k1_hist(nodes_int)
    bases, deg2_2d = _k2a(hist)
    perm = _k3_rank(nodes_int, bases)
    out_flat = _k4_gather(attr_flat, perm)
    out_edge_attr = out_flat.reshape(S, D_EDGE)

    deg2 = deg2_2d.reshape(-1)[:N, None]
    ns = deg2 * x
    n_pad = ((N + BLK - 1) // BLK) * BLK
    ns_p = jnp.pad(ns, ((0, n_pad - N), (0, 0)))
    cs = _k2c_cumsum(ns_p)
    out_x = cs[:N] / deg2
    return (out_x, out_edge_attr)


# speedup vs baseline: 5.0600x; 1.8284x over previous
import functools

import jax
import jax.numpy as jnp
from jax import lax
from jax.experimental import pallas as pl
from jax.experimental.pallas import tpu as pltpu
from jax.experimental.pallas import tpu_sc as plsc

N = 10000
E = 160000
S = 2 * E
D_EDGE = 16
NW = 32
PW = S // NW
NA, NB = 80, 128
CH = 400
BLK = 256

_mesh = plsc.VectorSubcoreMesh(core_axis_name="c", subcore_axis_name="s")
_sc_params = pltpu.CompilerParams(needs_layout_passes=False)


def _wid():
    return lax.axis_index("s") * 2 + lax.axis_index("c")


@functools.partial(
    pl.kernel, mesh=_mesh,
    out_type=jax.ShapeDtypeStruct((NW, NA, NB), jnp.int32),
    compiler_params=_sc_params,
    scratch_types=[
        pltpu.VMEM((PW,), jnp.int32),
        pltpu.VMEM((NA, NB), jnp.int32),
    ],
)
def _k1_hist(nodes_hbm, hist_hbm, nd_v, h_v):
    w = _wid()
    pltpu.sync_copy(nodes_hbm.at[pl.ds(w * PW, PW)], nd_v)
    zeros = jnp.zeros((16,), jnp.int32)

    def zbody(m, _):
        h_v[m >> 3, pl.ds((m & 7) * 16, 16)] = zeros
        return 0

    lax.fori_loop(0, NA * 8, zbody, 0, unroll=4)
    ones = jnp.ones((16,), jnp.int32)

    def body(t, _):
        nd = nd_v[pl.ds(t * 16, 16)]
        plsc.addupdate_scatter(h_v.at[:, :], [nd >> 7, nd & 127], ones)
        return 0

    lax.fori_loop(0, PW // 16, body, 0, unroll=4)
    pltpu.sync_copy(h_v, hist_hbm.at[w])


def _k2a_body(h_ref, bases_ref, deg2_ref, acc, offs, deg2s):
    i = pl.program_id(0)

    @pl.when(i == 0)
    def _():
        acc[...] = jnp.zeros_like(acc)
        offs[...] = jnp.zeros_like(offs)
        deg2s[...] = jnp.zeros_like(deg2s)

    h = h_ref[0].astype(jnp.float32)

    @pl.when(i < NW)
    def _():
        bases_ref[0] = jnp.zeros((NA, NB), jnp.int32)
        acc[...] = acc[...] + h

        @pl.when(i == NW - 1)
        def _():
            a = acc[...]
            r1 = lax.broadcasted_iota(jnp.int32, (NB, NB), 0)
            c1 = lax.broadcasted_iota(jnp.int32, (NB, NB), 1)
            tri_u = (r1 <= c1).astype(jnp.float32)
            incl = jnp.dot(a, tri_u, preferred_element_type=jnp.float32, precision=lax.Precision.HIGHEST)
            rowsum = jnp.sum(a, axis=1, keepdims=True)
            r2 = lax.broadcasted_iota(jnp.int32, (NA, NA), 0)
            c2 = lax.broadcasted_iota(jnp.int32, (NA, NA), 1)
            tri_l = (r2 > c2).astype(jnp.float32)
            rowpre = jnp.dot(tri_l, rowsum, preferred_element_type=jnp.float32, precision=lax.Precision.HIGHEST)
            offs[...] = incl - a + rowpre
            deg2s[...] = jnp.maximum(a * a, 1.0)
            acc[...] = jnp.zeros_like(acc)

    @pl.when(i >= NW)
    def _():
        bases_ref[0] = (offs[...] + acc[...]).astype(jnp.int32)
        acc[...] = acc[...] + h

    deg2_ref[...] = deg2s[...]


def _k2a(hist):
    return pl.pallas_call(
        _k2a_body,
        grid=(2 * NW,),
        in_specs=[pl.BlockSpec((1, NA, NB), lambda i: (i % NW, 0, 0))],
        out_specs=[
            pl.BlockSpec((1, NA, NB), lambda i: (i % NW, 0, 0)),
            pl.BlockSpec((NA, NB), lambda i: (0, 0)),
        ],
        out_shape=[
            jax.ShapeDtypeStruct((NW, NA, NB), jnp.int32),
            jax.ShapeDtypeStruct((NA, NB), jnp.float32),
        ],
        scratch_shapes=[
            pltpu.VMEM((NA, NB), jnp.float32),
            pltpu.VMEM((NA, NB), jnp.float32),
            pltpu.VMEM((NA, NB), jnp.float32),
        ],
    )(hist)


def _k2c_body(ns_ref, out_ref, carry_ref):
    i = pl.program_id(0)

    @pl.when(i == 0)
    def _():
        carry_ref[...] = jnp.zeros_like(carry_ref)

    blk = ns_ref[...]
    r = lax.broadcasted_iota(jnp.int32, (BLK, BLK), 0)
    c = lax.broadcasted_iota(jnp.int32, (BLK, BLK), 1)
    tri = (r >= c).astype(jnp.float32)
    cs = jnp.dot(tri, blk, preferred_element_type=jnp.float32, precision=lax.Precision.HIGHEST) + carry_ref[...]
    out_ref[...] = cs
    carry_ref[...] = cs[BLK - 1:BLK, :]


def _k2c_cumsum(ns):
    n_pad, d = ns.shape
    return pl.pallas_call(
        _k2c_body,
        grid=(n_pad // BLK,),
        in_specs=[pl.BlockSpec((BLK, d), lambda i: (i, 0))],
        out_specs=pl.BlockSpec((BLK, d), lambda i: (i, 0)),
        out_shape=jax.ShapeDtypeStruct((n_pad, d), jnp.float32),
        scratch_shapes=[pltpu.VMEM((1, d), jnp.float32)],
    )(ns)


TPS = S // 16


@functools.partial(
    pl.kernel, mesh=_mesh,
    out_type=jax.ShapeDtypeStruct((2 * S,), jnp.int32),
    compiler_params=_sc_params,
    scratch_types=[
        pltpu.VMEM((PW,), jnp.int32),
        pltpu.VMEM((NA, NB), jnp.int32),
        pltpu.VMEM((PW,), jnp.int32),
        pltpu.VMEM((PW,), jnp.int32),
        pltpu.VMEM((TPS,), jnp.int32),
        pltpu.VMEM_SHARED((S,), jnp.int32),
        pltpu.SemaphoreType.DMA,
    ],
)
def _k3_rank(nodes_hbm, bases_hbm, perm2_hbm, nd_v, cnt_v, pos_v, eid_v,
             sp_v, shared, sem):
    c = lax.axis_index("c")
    sid = lax.axis_index("s")
    w = sid * 2 + c
    zeros = jnp.zeros((16,), jnp.int32)

    def zbody(m, _):
        sp_v[pl.ds(m * 16, 16)] = zeros
        return 0

    lax.fori_loop(0, TPS // 16, zbody, 0, unroll=4)
    pltpu.sync_copy(sp_v, shared.at[pl.ds(sid * TPS, TPS)])
    pltpu.sync_copy(nodes_hbm.at[pl.ds(w * PW, PW)], nd_v)
    pltpu.sync_copy(bases_hbm.at[w], cnt_v)
    plsc.subcore_barrier()
    lanes = lax.iota(jnp.int32, 16)
    g0 = w * PW

    def body(t, _):
        sl = pl.ds(t * 16, 16)
        nd = nd_v[sl]
        hi = nd >> 7
        lo = nd & 127
        occ, last = plsc.scan_count(nd)
        base = plsc.load_gather(cnt_v.at[:, :], [hi, lo])
        pos_v[sl] = base + occ - 1
        plsc.store_scatter(cnt_v.at[:, :], [hi, lo], base + occ, mask=last)
        eid_v[sl] = (g0 + t * 16 + lanes) >> 1
        return 0

    lax.fori_loop(0, PW // 16, body, 0, unroll=2)
    pltpu.sync_copy(eid_v, shared.at[pos_v], add=True)
    plsc.subcore_barrier()
    pltpu.sync_copy(shared.at[pl.ds(sid * TPS, TPS)], sp_v)
    pltpu.sync_copy(sp_v, perm2_hbm.at[pl.ds(c * S + sid * TPS, TPS)])


NCH = PW // CH


@functools.partial(
    pl.kernel, mesh=_mesh,
    out_type=jax.ShapeDtypeStruct((S * D_EDGE,), jnp.float32),
    compiler_params=_sc_params,
    scratch_types=[
        pltpu.VMEM((PW,), jnp.int32),
        pltpu.VMEM((PW,), jnp.int32),
        pltpu.VMEM((CH, 128), jnp.float32),
        pltpu.VMEM((CH, 128), jnp.float32),
        pltpu.VMEM((CH * D_EDGE,), jnp.float32),
        pltpu.SemaphoreType.DMA,
        pltpu.SemaphoreType.DMA,
    ],
)
def _k4_gather(dup_hbm, perm2_hbm, out_hbm, pa_v, pb_v, st0, st1, cmp_v,
               sem0, sem1):
    w = _wid()
    base = w * PW
    pltpu.sync_copy(perm2_hbm.at[pl.ds(base, PW)], pa_v)
    pltpu.sync_copy(perm2_hbm.at[pl.ds(S + base, PW)], pb_v)

    def mbody(t, _):
        sl = pl.ds(t * 16, 16)
        pa_v[sl] = pa_v[sl] + pb_v[sl]
        return 0

    lax.fori_loop(0, PW // 16, mbody, 0, unroll=4)

    def compact_out(st, c):
        def rbody(j, _):
            cmp_v[pl.ds(j * D_EDGE, D_EDGE)] = st[j, pl.ds(0, D_EDGE)]
            return 0

        lax.fori_loop(0, CH, rbody, 0, unroll=8)
        pltpu.sync_copy(
            cmp_v, out_hbm.at[pl.ds((base + c * CH) * D_EDGE, CH * D_EDGE)])

    pltpu.async_copy(dup_hbm.at[pa_v.at[pl.ds(0, CH)]], st0, sem0)

    def chunk_body(k, _):
        c0 = k * 2
        pltpu.make_async_copy(dup_hbm.at[pa_v.at[pl.ds(0, CH)]], st0, sem0).wait()

        @pl.when(c0 + 1 < NCH)
        def _():
            pltpu.async_copy(
                dup_hbm.at[pa_v.at[pl.ds((c0 + 1) * CH, CH)]], st1, sem1)

        compact_out(st0, c0)

        @pl.when(c0 + 1 < NCH)
        def _():
            pltpu.make_async_copy(
                dup_hbm.at[pa_v.at[pl.ds(0, CH)]], st1, sem1).wait()

            @pl.when(c0 + 2 < NCH)
            def _():
                pltpu.async_copy(
                    dup_hbm.at[pa_v.at[pl.ds((c0 + 2) * CH, CH)]], st0, sem0)

            compact_out(st1, c0 + 1)

        return 0

    lax.fori_loop(0, (NCH + 1) // 2, chunk_body, 0)


def kernel(x, edge_index, edge_attr):
    nodes_int = edge_index.astype(jnp.int32).T.reshape(-1)
    attr_flat = edge_attr.reshape(-1)

    dup = jnp.repeat(edge_attr, 8, axis=0).reshape(E, 128)

    hist = _k1_hist(nodes_int)
    bases, deg2_2d = _k2a(hist)
    perm2 = _k3_rank(nodes_int, bases)
    out_flat = _k4_gather(dup, perm2)
    out_edge_attr = out_flat.reshape(S, D_EDGE)

    deg2 = deg2_2d.reshape(-1)[:N, None]
    ns = deg2 * x
    n_pad = ((N + BLK - 1) // BLK) * BLK
    ns_p = jnp.pad(ns, ((0, n_pad - N), (0, 0)))
    cs = _k2c_cumsum(ns_p)
    out_x = cs[:N] / deg2
    return (out_x, out_edge_attr)


# speedup vs baseline: 6.1949x; 1.2243x over previous
import functools

import jax
import jax.numpy as jnp
from jax import lax
from jax.experimental import pallas as pl
from jax.experimental.pallas import tpu as pltpu
from jax.experimental.pallas import tpu_sc as plsc

N = 10000
E = 160000
S = 2 * E
D_EDGE = 16
NW = 32
PW = S // NW
NA, NB = 80, 128
CH = 400
BLK = 256

_mesh = plsc.VectorSubcoreMesh(core_axis_name="c", subcore_axis_name="s")
_sc_params = pltpu.CompilerParams(needs_layout_passes=False)


def _wid():
    return lax.axis_index("s") * 2 + lax.axis_index("c")


def _interleave(ei_hbm, w, nd_v, u_v, v_v):
    e0 = w * (PW // 2)
    pltpu.sync_copy(ei_hbm.at[pl.ds(e0, PW // 2 + 8)], u_v)
    pltpu.sync_copy(ei_hbm.at[pl.ds(E + e0, PW // 2 + 8)], v_v)
    lanes = lax.iota(jnp.int32, 16)

    def ibody(t, _):
        sl = pl.ds(t * 16, 16)
        tgt = (t * 16 + lanes) * 2
        plsc.store_scatter(nd_v.at[:], [tgt], u_v[sl])
        plsc.store_scatter(nd_v.at[:], [tgt + 1], v_v[sl])
        return 0

    lax.fori_loop(0, (PW // 2 + 15) // 16, ibody, 0, unroll=4)


@functools.partial(
    pl.kernel, mesh=_mesh,
    out_type=jax.ShapeDtypeStruct((NW, NA, NB), jnp.int32),
    compiler_params=_sc_params,
    scratch_types=[
        pltpu.VMEM((PW + 16,), jnp.int32),
        pltpu.VMEM((NA, NB), jnp.int32),
        pltpu.VMEM((PW // 2 + 8,), jnp.int32),
        pltpu.VMEM((PW // 2 + 8,), jnp.int32),
    ],
)
def _k1_hist(ei_hbm, hist_hbm, nd_v, h_v, u_v, v_v):
    w = _wid()
    _interleave(ei_hbm, w, nd_v, u_v, v_v)
    zeros = jnp.zeros((16,), jnp.int32)

    def zbody(m, _):
        h_v[m >> 3, pl.ds((m & 7) * 16, 16)] = zeros
        return 0

    lax.fori_loop(0, NA * 8, zbody, 0, unroll=4)
    ones = jnp.ones((16,), jnp.int32)

    def body(t, _):
        nd = nd_v[pl.ds(t * 16, 16)]
        plsc.addupdate_scatter(h_v.at[:, :], [nd >> 7, nd & 127], ones)
        return 0

    lax.fori_loop(0, PW // 16, body, 0, unroll=4)
    pltpu.sync_copy(h_v, hist_hbm.at[w])


def _k2a_body(h_ref, bases_ref, deg2_ref, acc, offs, deg2s):
    i = pl.program_id(0)

    @pl.when(i == 0)
    def _():
        acc[...] = jnp.zeros_like(acc)
        offs[...] = jnp.zeros_like(offs)
        deg2s[...] = jnp.zeros_like(deg2s)

    h = h_ref[0].astype(jnp.float32)

    @pl.when(i < NW)
    def _():
        bases_ref[0] = jnp.zeros((NA, NB), jnp.int32)
        acc[...] = acc[...] + h

        @pl.when(i == NW - 1)
        def _():
            a = acc[...]
            r1 = lax.broadcasted_iota(jnp.int32, (NB, NB), 0)
            c1 = lax.broadcasted_iota(jnp.int32, (NB, NB), 1)
            tri_u = (r1 <= c1).astype(jnp.float32)
            incl = jnp.dot(a, tri_u, preferred_element_type=jnp.float32, precision=lax.Precision.HIGHEST)
            rowsum = jnp.sum(a, axis=1, keepdims=True)
            r2 = lax.broadcasted_iota(jnp.int32, (NA, NA), 0)
            c2 = lax.broadcasted_iota(jnp.int32, (NA, NA), 1)
            tri_l = (r2 > c2).astype(jnp.float32)
            rowpre = jnp.dot(tri_l, rowsum, preferred_element_type=jnp.float32, precision=lax.Precision.HIGHEST)
            offs[...] = incl - a + rowpre
            deg2s[...] = jnp.maximum(a * a, 1.0)
            acc[...] = jnp.zeros_like(acc)

    @pl.when(i >= NW)
    def _():
        bases_ref[0] = (offs[...] + acc[...]).astype(jnp.int32)
        acc[...] = acc[...] + h

    deg2_ref[...] = deg2s[...]


def _k2a(hist):
    return pl.pallas_call(
        _k2a_body,
        grid=(2 * NW,),
        in_specs=[pl.BlockSpec((1, NA, NB), lambda i: (i % NW, 0, 0))],
        out_specs=[
            pl.BlockSpec((1, NA, NB), lambda i: (i % NW, 0, 0)),
            pl.BlockSpec((NA, NB), lambda i: (0, 0)),
        ],
        out_shape=[
            jax.ShapeDtypeStruct((NW, NA, NB), jnp.int32),
            jax.ShapeDtypeStruct((NA, NB), jnp.float32),
        ],
        scratch_shapes=[
            pltpu.VMEM((NA, NB), jnp.float32),
            pltpu.VMEM((NA, NB), jnp.float32),
            pltpu.VMEM((NA, NB), jnp.float32),
        ],
    )(hist)


def _k2c_body(ns_ref, out_ref, carry_ref):
    i = pl.program_id(0)

    @pl.when(i == 0)
    def _():
        carry_ref[...] = jnp.zeros_like(carry_ref)

    blk = ns_ref[...]
    r = lax.broadcasted_iota(jnp.int32, (BLK, BLK), 0)
    c = lax.broadcasted_iota(jnp.int32, (BLK, BLK), 1)
    tri = (r >= c).astype(jnp.float32)
    cs = jnp.dot(tri, blk, preferred_element_type=jnp.float32) + carry_ref[...]
    out_ref[...] = cs
    carry_ref[...] = cs[BLK - 1:BLK, :]


def _k2c_cumsum(ns):
    n_pad, d = ns.shape
    return pl.pallas_call(
        _k2c_body,
        grid=(n_pad // BLK,),
        in_specs=[pl.BlockSpec((BLK, d), lambda i: (i, 0))],
        out_specs=pl.BlockSpec((BLK, d), lambda i: (i, 0)),
        out_shape=jax.ShapeDtypeStruct((n_pad, d), jnp.float32),
        scratch_shapes=[pltpu.VMEM((1, d), jnp.float32)],
    )(ns)


TPS = S // 16


@functools.partial(
    pl.kernel, mesh=_mesh,
    out_type=jax.ShapeDtypeStruct((2 * S,), jnp.int32),
    compiler_params=_sc_params,
    scratch_types=[
        pltpu.VMEM((PW + 16,), jnp.int32),
        pltpu.VMEM((NA, NB), jnp.int32),
        pltpu.VMEM((PW,), jnp.int32),
        pltpu.VMEM((PW,), jnp.int32),
        pltpu.VMEM((TPS,), jnp.int32),
        pltpu.VMEM((PW // 2 + 8,), jnp.int32),
        pltpu.VMEM((PW // 2 + 8,), jnp.int32),
        pltpu.VMEM_SHARED((S,), jnp.int32),
        pltpu.SemaphoreType.DMA,
    ],
)
def _k3_rank(ei_hbm, bases_hbm, perm2_hbm, nd_v, cnt_v, pos_v, eid_v,
             sp_v, u_v, v_v, shared, sem):
    c = lax.axis_index("c")
    sid = lax.axis_index("s")
    w = sid * 2 + c
    zeros = jnp.zeros((16,), jnp.int32)

    def zbody(m, _):
        sp_v[pl.ds(m * 16, 16)] = zeros
        return 0

    lax.fori_loop(0, TPS // 16, zbody, 0, unroll=4)
    pltpu.sync_copy(sp_v, shared.at[pl.ds(sid * TPS, TPS)])
    _interleave(ei_hbm, w, nd_v, u_v, v_v)
    pltpu.sync_copy(bases_hbm.at[w], cnt_v)
    plsc.subcore_barrier()
    lanes = lax.iota(jnp.int32, 16)
    g0 = w * PW

    def body(t, _):
        sl = pl.ds(t * 16, 16)
        nd = nd_v[sl]
        hi = nd >> 7
        lo = nd & 127
        occ, last = plsc.scan_count(nd)
        base = plsc.load_gather(cnt_v.at[:, :], [hi, lo])
        pos_v[sl] = base + occ - 1
        plsc.store_scatter(cnt_v.at[:, :], [hi, lo], base + occ, mask=last)
        eid_v[sl] = (g0 + t * 16 + lanes) >> 1
        return 0

    lax.fori_loop(0, PW // 16, body, 0, unroll=2)
    pltpu.sync_copy(eid_v, shared.at[pos_v], add=True)
    plsc.subcore_barrier()
    pltpu.sync_copy(shared.at[pl.ds(sid * TPS, TPS)], sp_v)
    pltpu.sync_copy(sp_v, perm2_hbm.at[pl.ds(c * S + sid * TPS, TPS)])


NCH = PW // CH


@functools.partial(
    pl.kernel, mesh=_mesh,
    out_type=jax.ShapeDtypeStruct((S * D_EDGE,), jnp.float32),
    compiler_params=_sc_params,
    scratch_types=[
        pltpu.VMEM((PW,), jnp.int32),
        pltpu.VMEM((PW,), jnp.int32),
        pltpu.VMEM((CH, 128), jnp.float32),
        pltpu.VMEM((CH, 128), jnp.float32),
        pltpu.VMEM((CH * D_EDGE,), jnp.float32),
        pltpu.SemaphoreType.DMA,
        pltpu.SemaphoreType.DMA,
    ],
)
def _k4_gather(dup_hbm, perm2_hbm, out_hbm, pa_v, pb_v, st0, st1, cmp_v,
               sem0, sem1):
    w = _wid()
    base = w * PW
    pltpu.sync_copy(perm2_hbm.at[pl.ds(base, PW)], pa_v)
    pltpu.sync_copy(perm2_hbm.at[pl.ds(S + base, PW)], pb_v)

    def mbody(t, _):
        sl = pl.ds(t * 16, 16)
        pa_v[sl] = pa_v[sl] + pb_v[sl]
        return 0

    lax.fori_loop(0, PW // 16, mbody, 0, unroll=4)

    def compact_out(st, c):
        def rbody(j, _):
            cmp_v[pl.ds(j * D_EDGE, D_EDGE)] = st[j, pl.ds(0, D_EDGE)]
            return 0

        lax.fori_loop(0, CH, rbody, 0, unroll=8)
        pltpu.sync_copy(
            cmp_v, out_hbm.at[pl.ds((base + c * CH) * D_EDGE, CH * D_EDGE)])

    pltpu.async_copy(dup_hbm.at[pa_v.at[pl.ds(0, CH)]], st0, sem0)

    def chunk_body(k, _):
        c0 = k * 2
        pltpu.make_async_copy(dup_hbm.at[pa_v.at[pl.ds(0, CH)]], st0, sem0).wait()

        @pl.when(c0 + 1 < NCH)
        def _():
            pltpu.async_copy(
                dup_hbm.at[pa_v.at[pl.ds((c0 + 1) * CH, CH)]], st1, sem1)

        compact_out(st0, c0)

        @pl.when(c0 + 1 < NCH)
        def _():
            pltpu.make_async_copy(
                dup_hbm.at[pa_v.at[pl.ds(0, CH)]], st1, sem1).wait()

            @pl.when(c0 + 2 < NCH)
            def _():
                pltpu.async_copy(
                    dup_hbm.at[pa_v.at[pl.ds((c0 + 2) * CH, CH)]], st0, sem0)

            compact_out(st1, c0 + 1)

        return 0

    lax.fori_loop(0, (NCH + 1) // 2, chunk_body, 0)


def kernel(x, edge_index, edge_attr):
    ei_flat = jnp.pad(edge_index.astype(jnp.int32).reshape(-1), (0, 16))

    dup = jnp.repeat(edge_attr, 8, axis=0).reshape(E, 128)

    hist = _k1_hist(ei_flat)
    bases, deg2_2d = _k2a(hist)
    perm2 = _k3_rank(ei_flat, bases)
    out_flat = _k4_gather(dup, perm2)
    out_edge_attr = out_flat.reshape(S, D_EDGE)

    deg2 = deg2_2d.reshape(-1)[:N, None]
    ns = deg2 * x
    n_pad = ((N + BLK - 1) // BLK) * BLK
    ns_p = jnp.pad(ns, ((0, n_pad - N), (0, 0)))
    cs = _k2c_cumsum(ns_p)
    out_x = cs[:N] / deg2
    return (out_x, out_edge_attr)
